# baseline (speedup 1.0000x reference)
import functools

import jax
import jax.numpy as jnp
from jax import lax
from jax.experimental import pallas as pl
from jax.experimental.pallas import tpu as pltpu

B, N = 32, 2048
NUM_GROUP, GROUP_SIZE = 128, 32
TRANS_DIM = 384
HIDDEN = 256
NUM_CLASSES = 40
FILTER_K = 20


def _fps_kernel(xt_ref, cent_ref):
    xcat = jnp.concatenate([xt_ref[0], xt_ref[1], xt_ref[2]],
                           axis=0)
    iota_n = lax.broadcasted_iota(jnp.int32, (B, N), 1)
    iota3 = lax.broadcasted_iota(jnp.int32, (3 * B, N), 1)
    dist0 = jnp.full((B, N), 1e10, dtype=jnp.float32)
    far0 = jnp.zeros((B, 1), dtype=jnp.int32)

    def body(i, st):
        dist, far = st
        m3 = (iota3 == jnp.concatenate([far, far, far], axis=0)
              ).astype(jnp.float32)
        pcat = jnp.sum(m3 * xcat, axis=1, keepdims=True)
        cent_ref[pl.ds(i, 1)] = jnp.concatenate(
            [pcat[0:B].T, pcat[B:2 * B].T, pcat[2 * B:].T],
            axis=0)[None]
        t = xcat - pcat
        t = t * t
        d = t[0:B] + t[B:2 * B] + t[2 * B:]
        dist = jnp.minimum(dist, d)
        far = jnp.argmax(dist, axis=1)[:, None]
        return dist, far

    lax.fori_loop(0, NUM_GROUP, body, (dist0, far0))


_NB = 8


def _group_kernel(xr_ref, xt_ref, cent_ref, wt_ref, bp_ref, out_ref):
    wt = wt_ref[...]
    R = _NB * NUM_GROUP
    d_blocks = []
    for b in range(_NB):
        xb = xt_ref[b]
        sqn = jnp.sum(xb * xb, axis=0, keepdims=True)
        cb = cent_ref[b]
        sqc = jnp.sum(cb * cb, axis=1, keepdims=True)
        g = jnp.dot(cb, xb, preferred_element_type=jnp.float32)
        d_blocks.append(sqc - 2.0 * g + sqn)
    d = jnp.concatenate(d_blocks, axis=0)
    iota_n = lax.broadcasted_iota(jnp.int32, (R, N), 1)

    def body(s, st):
        dcur, mmax = st
        am = jnp.argmin(dcur, axis=1)
        oh = iota_n == am[:, None]
        ohf = oh.astype(jnp.float32)
        xgs = []
        for b in range(_NB):
            ohb = ohf[b * NUM_GROUP:(b + 1) * NUM_GROUP]
            xgs.append(jnp.dot(ohb, xr_ref[b],
                               preferred_element_type=jnp.float32))
        xg = jnp.concatenate(xgs, axis=0)
        u = jnp.dot(xg, wt, preferred_element_type=jnp.float32)
        mmax = jnp.maximum(mmax, u)
        dcur = jnp.where(oh, jnp.float32(jnp.inf), dcur)
        return dcur, mmax

    mmax0 = jnp.full((R, TRANS_DIM), -jnp.inf, dtype=jnp.float32)
    _, mmax = lax.fori_loop(0, GROUP_SIZE, body, (d, mmax0))
    cb_all = jnp.concatenate([cent_ref[b] for b in range(_NB)], axis=0)
    vb = jnp.dot(cb_all, wt, preferred_element_type=jnp.float32)
    tok = jnp.maximum(mmax - vb + bp_ref[...], 0.0)
    tok4 = tok.reshape(_NB, NUM_GROUP, TRANS_DIM)
    out_ref[...] = jnp.max(tok4, axis=1, keepdims=True)


def _ent_lab(p):
    mx = jnp.max(p, axis=1, keepdims=True)
    e = jnp.exp(p - mx)
    s = jnp.sum(e, axis=1, keepdims=True)
    logp = (p - mx) - jnp.log(s)
    ent = -jnp.sum((e / s) * logp, axis=1, keepdims=True)
    am = jnp.argmax(p, axis=1)
    return ent, am


def _head_kernel(tmax_ref, cls_ref, w1t_ref, b1_ref, w2_ref, w2t_ref,
                 b2_ref, out_ref):
    tmax = tmax_ref[...]
    cls_h = jnp.dot(cls_ref[...], w1t_ref[:TRANS_DIM],
                    preferred_element_type=jnp.float32)
    tm_h = jnp.dot(tmax, w1t_ref[TRANS_DIM:],
                   preferred_element_type=jnp.float32)
    h = jnp.maximum(tm_h + cls_h + b1_ref[...], 0.0)
    w2 = w2_ref[...]
    w2t = w2t_ref[...]
    b2 = b2_ref[...]
    p = jnp.dot(h, w2t, preferred_element_type=jnp.float32) + b2
    warm_p = jnp.dot(w2, w2t, preferred_element_type=jnp.float32) + b2
    went, wam = _ent_lab(warm_p)
    ent, am = _ent_lab(p)
    iota_c = lax.broadcasted_iota(jnp.int32, (NUM_CLASSES + B, NUM_CLASSES), 1)
    ycls = jnp.concatenate([wam, am])[:, None]
    labels = (iota_c == ycls).astype(jnp.float32)
    ents = jnp.concatenate([went, ent], axis=0)
    jidx = lax.broadcasted_iota(jnp.int32, (NUM_CLASSES + B, 1), 0)
    same = lax.dot_general(labels, labels, (((1,), (1,)), ((), ())),
                           preferred_element_type=jnp.float32) > 0.5
    ents_t = jnp.transpose(ents)
    jidx_t = lax.broadcasted_iota(jnp.int32, (1, NUM_CLASSES + B), 1)
    less = (ents_t < ents) | ((ents_t == ents) & (jidx_t < jidx))
    rank = jnp.sum(jnp.where(same & less, 1.0, 0.0), axis=1, keepdims=True)
    inc = (rank < FILTER_K).astype(jnp.float32)
    supports = jnp.concatenate([w2, h], axis=0)
    nrm = jnp.sqrt(jnp.sum(supports * supports, axis=1, keepdims=True))
    s_norm = supports / jnp.maximum(nrm, 1e-12)
    sw = s_norm * inc
    weights = lax.dot_general(sw, labels, (((0,), (0,)), ((), ())),
                              preferred_element_type=jnp.float32)
    wn = jnp.sqrt(jnp.sum(weights * weights, axis=0, keepdims=True))
    w_norm = weights / jnp.maximum(wn, 1e-12)
    out_ref[...] = jnp.dot(h, w_norm, preferred_element_type=jnp.float32)


@functools.partial(jax.jit, static_argnames=("interpret",))
def _run(x, W_point, b_point, cls_token, W1, b1, W2, b2, interpret=False):
    xt = jnp.transpose(x, (2, 0, 1))
    cent = pl.pallas_call(
        _fps_kernel,
        out_shape=jax.ShapeDtypeStruct((NUM_GROUP, 3, B), jnp.float32),
        interpret=interpret,
    )(xt)
    cent_b = jnp.transpose(cent, (2, 0, 1))
    wt = jnp.transpose(W_point)
    tmax = pl.pallas_call(
        _group_kernel,
        grid=(B // _NB,),
        in_specs=[
            pl.BlockSpec((_NB, N, 3), lambda b: (b, 0, 0)),
            pl.BlockSpec((_NB, 3, N), lambda b: (b, 0, 0)),
            pl.BlockSpec((_NB, NUM_GROUP, 3), lambda b: (b, 0, 0)),
            pl.BlockSpec((3, TRANS_DIM), lambda b: (0, 0)),
            pl.BlockSpec((1, TRANS_DIM), lambda b: (0, 0)),
        ],
        out_specs=pl.BlockSpec((_NB, 1, TRANS_DIM), lambda b: (b, 0, 0)),
        out_shape=jax.ShapeDtypeStruct((B, 1, TRANS_DIM), jnp.float32),
        interpret=interpret,
    )(x, jnp.transpose(x, (0, 2, 1)), cent_b, wt, b_point[None, :])
    tmax = tmax[:, 0, :]
    out = pl.pallas_call(
        _head_kernel,
        out_shape=jax.ShapeDtypeStruct((B, NUM_CLASSES), jnp.float32),
        interpret=interpret,
    )(tmax, cls_token[None, :], jnp.transpose(W1), b1[None, :],
      W2, jnp.transpose(W2), b2[None, :])
    return out


def kernel(x, W_point, b_point, cls_token, W1, b1, W2, b2):
    return _run(x, W_point, b_point, cls_token, W1, b1, W2, b2)


# speedup vs baseline: 1.0068x; 1.0068x over previous
import functools

import jax
import jax.numpy as jnp
from jax import lax
from jax.experimental import pallas as pl
from jax.experimental.pallas import tpu as pltpu

B, N = 32, 2048
NUM_GROUP, GROUP_SIZE = 128, 32
TRANS_DIM = 384
HIDDEN = 256
NUM_CLASSES = 40
FILTER_K = 20


def _fps_kernel(xt_ref, cent_ref):
    x0 = xt_ref[0]
    x1 = xt_ref[1]
    x2 = xt_ref[2]
    iota_n = lax.broadcasted_iota(jnp.int32, (B, N), 1)
    dist0 = jnp.full((B, N), 1e10, dtype=jnp.float32)
    m0 = (iota_n == 0).astype(jnp.float32)

    def body(i, st):
        dist, m = st
        p0 = jnp.sum(m * x0, axis=1, keepdims=True)
        p1 = jnp.sum(m * x1, axis=1, keepdims=True)
        p2 = jnp.sum(m * x2, axis=1, keepdims=True)
        cent_ref[pl.ds(i, 1)] = jnp.concatenate(
            [p0.T, p1.T, p2.T], axis=0)[None]
        t0 = x0 - p0
        t1 = x1 - p1
        t2 = x2 - p2
        d = t0 * t0 + t1 * t1 + t2 * t2
        dist = jnp.minimum(dist, d)
        far = jnp.argmax(dist, axis=1)
        m = (iota_n == far[:, None]).astype(jnp.float32)
        return dist, m

    lax.fori_loop(0, NUM_GROUP, body, (dist0, m0))


_NB = 8


def _group_kernel(xr_ref, xt_ref, cent_ref, wt_ref, bp_ref, out_ref):
    wt = wt_ref[...]
    R = _NB * NUM_GROUP
    d_blocks = []
    for b in range(_NB):
        xb = xt_ref[b]
        sqn = jnp.sum(xb * xb, axis=0, keepdims=True)
        cb = cent_ref[b]
        sqc = jnp.sum(cb * cb, axis=1, keepdims=True)
        g = jnp.dot(cb, xb, preferred_element_type=jnp.float32)
        d_blocks.append(sqc - 2.0 * g + sqn)
    d = jnp.concatenate(d_blocks, axis=0)
    iota_n = lax.broadcasted_iota(jnp.int32, (R, N), 1)

    def body(s, st):
        dcur, mmax = st
        am = jnp.argmin(dcur, axis=1)
        oh = iota_n == am[:, None]
        ohf = oh.astype(jnp.float32)
        xgs = []
        for b in range(_NB):
            ohb = ohf[b * NUM_GROUP:(b + 1) * NUM_GROUP]
            xgs.append(jnp.dot(ohb, xr_ref[b],
                               preferred_element_type=jnp.float32))
        xg = jnp.concatenate(xgs, axis=0)
        u = jnp.dot(xg, wt, preferred_element_type=jnp.float32)
        mmax = jnp.maximum(mmax, u)
        dcur = jnp.where(oh, jnp.float32(jnp.inf), dcur)
        return dcur, mmax

    mmax0 = jnp.full((R, TRANS_DIM), -jnp.inf, dtype=jnp.float32)
    _, mmax = lax.fori_loop(0, GROUP_SIZE, body, (d, mmax0))
    cb_all = jnp.concatenate([cent_ref[b] for b in range(_NB)], axis=0)
    vb = jnp.dot(cb_all, wt, preferred_element_type=jnp.float32)
    tok = jnp.maximum(mmax - vb + bp_ref[...], 0.0)
    tok4 = tok.reshape(_NB, NUM_GROUP, TRANS_DIM)
    out_ref[...] = jnp.max(tok4, axis=1, keepdims=True)


def _ent_lab(p):
    mx = jnp.max(p, axis=1, keepdims=True)
    e = jnp.exp(p - mx)
    s = jnp.sum(e, axis=1, keepdims=True)
    logp = (p - mx) - jnp.log(s)
    ent = -jnp.sum((e / s) * logp, axis=1, keepdims=True)
    am = jnp.argmax(p, axis=1)
    return ent, am


def _head_kernel(tmax_ref, cls_ref, w1t_ref, b1_ref, w2_ref, w2t_ref,
                 b2_ref, out_ref):
    tmax = tmax_ref[...]
    cls_h = jnp.dot(cls_ref[...], w1t_ref[:TRANS_DIM],
                    preferred_element_type=jnp.float32)
    tm_h = jnp.dot(tmax, w1t_ref[TRANS_DIM:],
                   preferred_element_type=jnp.float32)
    h = jnp.maximum(tm_h + cls_h + b1_ref[...], 0.0)
    w2 = w2_ref[...]
    w2t = w2t_ref[...]
    b2 = b2_ref[...]
    p = jnp.dot(h, w2t, preferred_element_type=jnp.float32) + b2
    warm_p = jnp.dot(w2, w2t, preferred_element_type=jnp.float32) + b2
    went, wam = _ent_lab(warm_p)
    ent, am = _ent_lab(p)
    iota_c = lax.broadcasted_iota(jnp.int32, (NUM_CLASSES + B, NUM_CLASSES), 1)
    ycls = jnp.concatenate([wam, am])[:, None]
    labels = (iota_c == ycls).astype(jnp.float32)
    ents = jnp.concatenate([went, ent], axis=0)
    jidx = lax.broadcasted_iota(jnp.int32, (NUM_CLASSES + B, 1), 0)
    same = lax.dot_general(labels, labels, (((1,), (1,)), ((), ())),
                           preferred_element_type=jnp.float32) > 0.5
    ents_t = jnp.transpose(ents)
    jidx_t = lax.broadcasted_iota(jnp.int32, (1, NUM_CLASSES + B), 1)
    less = (ents_t < ents) | ((ents_t == ents) & (jidx_t < jidx))
    rank = jnp.sum(jnp.where(same & less, 1.0, 0.0), axis=1, keepdims=True)
    inc = (rank < FILTER_K).astype(jnp.float32)
    supports = jnp.concatenate([w2, h], axis=0)
    nrm = jnp.sqrt(jnp.sum(supports * supports, axis=1, keepdims=True))
    s_norm = supports / jnp.maximum(nrm, 1e-12)
    sw = s_norm * inc
    weights = lax.dot_general(sw, labels, (((0,), (0,)), ((), ())),
                              preferred_element_type=jnp.float32)
    wn = jnp.sqrt(jnp.sum(weights * weights, axis=0, keepdims=True))
    w_norm = weights / jnp.maximum(wn, 1e-12)
    out_ref[...] = jnp.dot(h, w_norm, preferred_element_type=jnp.float32)


@functools.partial(jax.jit, static_argnames=("interpret",))
def _run(x, W_point, b_point, cls_token, W1, b1, W2, b2, interpret=False):
    xt = jnp.transpose(x, (2, 0, 1))
    cent = pl.pallas_call(
        _fps_kernel,
        out_shape=jax.ShapeDtypeStruct((NUM_GROUP, 3, B), jnp.float32),
        interpret=interpret,
    )(xt)
    cent_b = jnp.transpose(cent, (2, 0, 1))
    wt = jnp.transpose(W_point)
    tmax = pl.pallas_call(
        _group_kernel,
        grid=(B // _NB,),
        in_specs=[
            pl.BlockSpec((_NB, N, 3), lambda b: (b, 0, 0)),
            pl.BlockSpec((_NB, 3, N), lambda b: (b, 0, 0)),
            pl.BlockSpec((_NB, NUM_GROUP, 3), lambda b: (b, 0, 0)),
            pl.BlockSpec((3, TRANS_DIM), lambda b: (0, 0)),
            pl.BlockSpec((1, TRANS_DIM), lambda b: (0, 0)),
        ],
        out_specs=pl.BlockSpec((_NB, 1, TRANS_DIM), lambda b: (b, 0, 0)),
        out_shape=jax.ShapeDtypeStruct((B, 1, TRANS_DIM), jnp.float32),
        interpret=interpret,
    )(x, jnp.transpose(x, (0, 2, 1)), cent_b, wt, b_point[None, :])
    tmax = tmax[:, 0, :]
    out = pl.pallas_call(
        _head_kernel,
        out_shape=jax.ShapeDtypeStruct((B, NUM_CLASSES), jnp.float32),
        interpret=interpret,
    )(tmax, cls_token[None, :], jnp.transpose(W1), b1[None, :],
      W2, jnp.transpose(W2), b2[None, :])
    return out


def kernel(x, W_point, b_point, cls_token, W1, b1, W2, b2):
    return _run(x, W_point, b_point, cls_token, W1, b1, W2, b2)
